# baseline (speedup 1.0000x reference)
import functools
import math

import jax
import jax.numpy as jnp
from jax import lax
from jax.experimental import pallas as pl
from jax.experimental.pallas import tpu as pltpu
from jax.experimental.pallas import tpu_sc as plsc

F0_BIN = 256
F0_MEL_MIN = 1127.0 * math.log(1.0 + 50.0 / 700.0)
F0_MEL_MAX = 1127.0 * math.log(1.0 + 1100.0 / 700.0)


def _index_body(t_ph, mel2ph_ref, f0_ref, uv_ref, energy_ref,
                gidx_ref, pidx_ref, eidx_ref, mask_ref):
    m = mel2ph_ref[...]
    f0 = f0_ref[...]
    uv = uv_ref[...]
    en = energy_ref[...]
    b_iota = lax.broadcasted_iota(jnp.int32, m.shape, 0)
    gidx_ref[...] = b_iota * t_ph + jnp.maximum(m - 1, 0)
    mask_ref[...] = jnp.broadcast_to(
        (m > 0).astype(jnp.float32)[:, :, None], m.shape + (16,))
    f0d = jnp.exp2(f0)
    f0d = jnp.where(uv > 0, 0.0, f0d)
    f0d = jnp.where(m == 0, 0.0, f0d)
    f0_mel = 1127.0 * jnp.log(1.0 + f0d / 700.0)
    f0_mel = jnp.where(
        f0_mel > 0,
        (f0_mel - F0_MEL_MIN) * (F0_BIN - 2) / (F0_MEL_MAX - F0_MEL_MIN) + 1.0,
        f0_mel)
    f0_mel = jnp.where(f0_mel <= 1.0, 1.0, f0_mel)
    f0_mel = jnp.where(f0_mel > F0_BIN - 1, float(F0_BIN - 1), f0_mel)
    pidx_ref[...] = (f0_mel + 0.5).astype(jnp.int32)
    e = jnp.minimum(jnp.floor(en * (256.0 / 4.0)), 255.0)
    eidx_ref[...] = jnp.maximum(e, 0.0).astype(jnp.int32)


def _compute_indices(mel2ph, f0, uv, energy, t_ph):
    B, T = mel2ph.shape
    i32 = jax.ShapeDtypeStruct((B, T), jnp.int32)
    f32x16 = jax.ShapeDtypeStruct((B, T, 16), jnp.float32)
    return pl.pallas_call(
        functools.partial(_index_body, t_ph),
        out_shape=[i32, i32, i32, f32x16],
    )(mel2ph, f0, uv, energy)


@functools.lru_cache(maxsize=None)
def _make_sc_kernel(rows_total, H, T_mel, NW):
    rpw = rows_total // NW
    NB = 64
    nblk = rpw // NB
    nseg = H // 16
    mesh = plsc.VectorSubcoreMesh(core_axis_name="c", subcore_axis_name="s")

    @functools.partial(
        pl.kernel,
        out_type=jax.ShapeDtypeStruct((rows_total, H), jnp.float32),
        mesh=mesh,
        scratch_types=[
            pltpu.VMEM((rpw,), jnp.int32),
            pltpu.VMEM((rpw,), jnp.int32),
            pltpu.VMEM((rpw,), jnp.int32),
            [pltpu.VMEM((NB, 16), jnp.float32) for _ in range(2)],
            [pltpu.VMEM((NB, H), jnp.float32) for _ in range(2)],
            [pltpu.VMEM((NB, H), jnp.float32) for _ in range(2)],
            [pltpu.VMEM((NB, H), jnp.float32) for _ in range(2)],
            pltpu.VMEM((16,), jnp.int32),
            pltpu.VMEM((16, H), jnp.float32),
            [pltpu.SemaphoreType.DMA for _ in range(2)],
            [pltpu.SemaphoreType.DMA for _ in range(2)],
        ],
    )
    def sck(hub_hbm, gidx_hbm, pidx_hbm, eidx_hbm, mask_hbm,
            wp_hbm, we_hbm, wspk_hbm, sid_hbm, out_hbm,
            gall, pall, eall, maskb, hubb, ppb, eeb,
            sidbuf, svbuf, semg, semst):
        wid = lax.axis_index("s") * 2 + lax.axis_index("c")
        base = wid * rpw
        b = base // T_mel
        pltpu.sync_copy(sid_hbm, sidbuf)
        pltpu.async_copy(wspk_hbm.at[sidbuf], svbuf, semg[0]).wait()
        pltpu.sync_copy(gidx_hbm.at[pl.ds(base, rpw)], gall)
        pltpu.sync_copy(pidx_hbm.at[pl.ds(base, rpw)], pall)
        pltpu.sync_copy(eidx_hbm.at[pl.ds(base, rpw)], eall)

        def issue(k, s):
            o = k * NB
            pltpu.async_copy(hub_hbm.at[gall.at[pl.ds(o, NB)]], hubb[s], semg[s])
            pltpu.async_copy(wp_hbm.at[pall.at[pl.ds(o, NB)]], ppb[s], semg[s])
            pltpu.async_copy(we_hbm.at[eall.at[pl.ds(o, NB)]], eeb[s], semg[s])
            pltpu.async_copy(mask_hbm.at[pl.ds(base + o, NB)], maskb[s], semg[s])

        def wait_gathers(k, s):
            o = k * NB
            pltpu.make_async_copy(hub_hbm.at[gall.at[pl.ds(o, NB)]], hubb[s], semg[s]).wait()
            pltpu.make_async_copy(wp_hbm.at[pall.at[pl.ds(o, NB)]], ppb[s], semg[s]).wait()
            pltpu.make_async_copy(we_hbm.at[eall.at[pl.ds(o, NB)]], eeb[s], semg[s]).wait()
            pltpu.make_async_copy(mask_hbm.at[pl.ds(base + o, NB)], maskb[s], semg[s]).wait()

        def wait_store(k, s):
            pltpu.make_async_copy(
                hubb[s], out_hbm.at[pl.ds(base + k * NB, NB)], semst[s]).wait()

        def compute(s):
            def row(i, c):
                maskv = maskb[s][i, :]
                for j in range(nseg):
                    sl = pl.ds(j * 16, 16)
                    acc = (hubb[s][i, sl] + ppb[s][i, sl] + eeb[s][i, sl]
                           + svbuf[b, sl])
                    hubb[s][i, sl] = acc * maskv
                return c
            lax.fori_loop(0, NB, row, 0)

        issue(0, 0)

        def step(k, carry):
            for u in range(2):
                kk = 2 * k + u
                s = u
                s2 = 1 - u

                @pl.when(kk + 1 < nblk)
                def _():
                    @pl.when(kk >= 1)
                    def _():
                        wait_store(kk - 1, s2)
                    issue(kk + 1, s2)

                wait_gathers(kk, s)
                compute(s)
                pltpu.async_copy(
                    hubb[s], out_hbm.at[pl.ds(base + kk * NB, NB)], semst[s])
            return carry

        lax.fori_loop(0, nblk // 2, step, 0)
        wait_store(nblk - 2, (nblk - 2) % 2)
        wait_store(nblk - 1, (nblk - 1) % 2)

    return sck


def kernel(hubert, mel2ph, spk_embed, f0, uv, energy, W_spk, W_pitch, W_energy):
    B, T_ph, H = hubert.shape
    T_mel = mel2ph.shape[1]
    gidx, pidx, eidx, mask = _compute_indices(mel2ph, f0, uv, energy, T_ph)
    hub2d = hubert.reshape(B * T_ph, H)
    rows_total = B * T_mel
    sck = _make_sc_kernel(rows_total, H, T_mel, 32)
    out = sck(hub2d, gidx.reshape(-1), pidx.reshape(-1), eidx.reshape(-1),
              mask.reshape(rows_total, 16), W_pitch, W_energy, W_spk,
              spk_embed)
    return out.reshape(B, T_mel, H)


# speedup vs baseline: 1.0039x; 1.0039x over previous
import functools
import math

import jax
import jax.numpy as jnp
from jax import lax
from jax.experimental import pallas as pl
from jax.experimental.pallas import tpu as pltpu
from jax.experimental.pallas import tpu_sc as plsc

F0_BIN = 256
F0_MEL_MIN = 1127.0 * math.log(1.0 + 50.0 / 700.0)
F0_MEL_MAX = 1127.0 * math.log(1.0 + 1100.0 / 700.0)


def _index_body(t_ph, mel2ph_ref, f0_ref, uv_ref, energy_ref,
                gidx_ref, pidx_ref, eidx_ref, mask_ref):
    m = mel2ph_ref[...]
    f0 = f0_ref[...]
    uv = uv_ref[...]
    en = energy_ref[...]
    b_iota = lax.broadcasted_iota(jnp.int32, m.shape, 0)
    gidx_ref[...] = b_iota * t_ph + jnp.maximum(m - 1, 0)
    mask_ref[...] = jnp.broadcast_to(
        (m > 0).astype(jnp.float32)[:, :, None], m.shape + (16,))
    f0d = jnp.exp2(f0)
    f0d = jnp.where(uv > 0, 0.0, f0d)
    f0d = jnp.where(m == 0, 0.0, f0d)
    f0_mel = 1127.0 * jnp.log(1.0 + f0d / 700.0)
    f0_mel = jnp.where(
        f0_mel > 0,
        (f0_mel - F0_MEL_MIN) * (F0_BIN - 2) / (F0_MEL_MAX - F0_MEL_MIN) + 1.0,
        f0_mel)
    f0_mel = jnp.where(f0_mel <= 1.0, 1.0, f0_mel)
    f0_mel = jnp.where(f0_mel > F0_BIN - 1, float(F0_BIN - 1), f0_mel)
    pidx_ref[...] = (f0_mel + 0.5).astype(jnp.int32)
    e = jnp.minimum(jnp.floor(en * (256.0 / 4.0)), 255.0)
    eidx_ref[...] = jnp.maximum(e, 0.0).astype(jnp.int32)


def _compute_indices(mel2ph, f0, uv, energy, t_ph):
    B, T = mel2ph.shape
    i32 = jax.ShapeDtypeStruct((B, T), jnp.int32)
    f32x16 = jax.ShapeDtypeStruct((B, T, 16), jnp.float32)
    return pl.pallas_call(
        functools.partial(_index_body, t_ph),
        out_shape=[i32, i32, i32, f32x16],
    )(mel2ph, f0, uv, energy)


@functools.lru_cache(maxsize=None)
def _make_sc_kernel(rows_total, H, T_mel, NW):
    rpw = rows_total // NW
    NB = 64
    nblk = rpw // NB
    nseg = H // 16
    mesh = plsc.VectorSubcoreMesh(core_axis_name="c", subcore_axis_name="s")

    @functools.partial(
        pl.kernel,
        out_type=jax.ShapeDtypeStruct((rows_total, H), jnp.float32),
        mesh=mesh,
        scratch_types=[
            pltpu.VMEM((rpw,), jnp.int32),
            pltpu.VMEM((rpw,), jnp.int32),
            pltpu.VMEM((rpw,), jnp.int32),
            pltpu.VMEM((NB, 16), jnp.float32),
            pltpu.VMEM((NB, H), jnp.float32),
            pltpu.VMEM((16,), jnp.int32),
            pltpu.VMEM((16, H), jnp.float32),
            pltpu.SemaphoreType.DMA,
            pltpu.SemaphoreType.DMA,
        ],
    )
    def sck(hub_hbm, gidx_hbm, pidx_hbm, eidx_hbm, mask_hbm,
            wp_hbm, we_hbm, wspk_hbm, sid_hbm, out_hbm,
            gall, pall, eall, maskb, hubb, sidbuf, svbuf, semg, semst):
        wid = lax.axis_index("s") * 2 + lax.axis_index("c")
        base = wid * rpw
        b = base // T_mel
        pltpu.sync_copy(sid_hbm, sidbuf)
        pltpu.async_copy(wspk_hbm.at[sidbuf], svbuf, semg).wait()
        pltpu.sync_copy(gidx_hbm.at[pl.ds(base, rpw)], gall)
        pltpu.sync_copy(pidx_hbm.at[pl.ds(base, rpw)], pall)
        pltpu.sync_copy(eidx_hbm.at[pl.ds(base, rpw)], eall)

        def blk(k, carry):
            o = k * NB
            pltpu.async_copy(hub_hbm.at[gall.at[pl.ds(o, NB)]], hubb, semg).wait()
            pltpu.async_copy(wp_hbm.at[pall.at[pl.ds(o, NB)]], hubb, semg,
                             add=True).wait()
            pltpu.async_copy(we_hbm.at[eall.at[pl.ds(o, NB)]], hubb, semg,
                             add=True).wait()
            pltpu.sync_copy(mask_hbm.at[pl.ds(base + o, NB)], maskb)

            spk_segs = [svbuf[b, pl.ds(j * 16, 16)] for j in range(nseg)]

            @plsc.parallel_loop(0, NB, step=1, unroll=2)
            def row(i):
                mv = maskb[i, :]
                for j in range(nseg):
                    sl = pl.ds(j * 16, 16)
                    hubb[i, sl] = (hubb[i, sl] + spk_segs[j]) * mv
            pltpu.sync_copy(hubb, out_hbm.at[pl.ds(base + o, NB)])
            return carry

        lax.fori_loop(0, nblk, blk, 0)

    return sck


def kernel(hubert, mel2ph, spk_embed, f0, uv, energy, W_spk, W_pitch, W_energy):
    B, T_ph, H = hubert.shape
    T_mel = mel2ph.shape[1]
    gidx, pidx, eidx, mask = _compute_indices(mel2ph, f0, uv, energy, T_ph)
    hub2d = hubert.reshape(B * T_ph, H)
    rows_total = B * T_mel
    sck = _make_sc_kernel(rows_total, H, T_mel, 32)
    out = sck(hub2d, gidx.reshape(-1), pidx.reshape(-1), eidx.reshape(-1),
              mask.reshape(rows_total, 16), W_pitch, W_energy, W_spk,
              spk_embed)
    return out.reshape(B, T_mel, H)


# speedup vs baseline: 14.3031x; 14.2472x over previous
import functools
import math

import jax
import jax.numpy as jnp
from jax import lax
from jax.experimental import pallas as pl
from jax.experimental.pallas import tpu as pltpu
from jax.experimental.pallas import tpu_sc as plsc

F0_BIN = 256
F0_MEL_MIN = 1127.0 * math.log(1.0 + 50.0 / 700.0)
F0_MEL_MAX = 1127.0 * math.log(1.0 + 1100.0 / 700.0)

NP_PAD = 384
NS_PAD = 128


@functools.lru_cache(maxsize=None)
def _make_sc_gather(rows_total, H, T_mel, T_ph, NW):
    rpw = rows_total // NW
    NB = 128
    nblk = rpw // NB
    mesh = plsc.VectorSubcoreMesh(core_axis_name="c", subcore_axis_name="s")

    @functools.partial(
        pl.kernel,
        out_type=jax.ShapeDtypeStruct((rows_total, H), jnp.float32),
        mesh=mesh,
        scratch_types=[
            pltpu.VMEM((rpw,), jnp.int32),
            [pltpu.VMEM((NB, H), jnp.float32) for _ in range(2)],
            [pltpu.SemaphoreType.DMA for _ in range(2)],
            [pltpu.SemaphoreType.DMA for _ in range(2)],
        ],
    )
    def sck(m2p_hbm, hub_hbm, out_hbm, gall, hubb, semg, semst):
        wid = lax.axis_index("s") * 2 + lax.axis_index("c")
        base = wid * rpw
        boff = (base // T_mel) * T_ph
        pltpu.sync_copy(m2p_hbm.at[pl.ds(base, rpw)], gall)

        @plsc.parallel_loop(0, rpw // 16, step=1, unroll=4)
        def fix(i):
            sl = pl.ds(i * 16, 16)
            m = gall[sl]
            gall[sl] = jnp.maximum(m - 1, 0) + boff

        def issue(k, s):
            pltpu.async_copy(hub_hbm.at[gall.at[pl.ds(k * NB, NB)]],
                             hubb[s], semg[s])

        def wait_gather(k, s):
            pltpu.make_async_copy(hub_hbm.at[gall.at[pl.ds(k * NB, NB)]],
                                  hubb[s], semg[s]).wait()

        def wait_store(k, s):
            pltpu.make_async_copy(
                hubb[s], out_hbm.at[pl.ds(base + k * NB, NB)], semst[s]).wait()

        issue(0, 0)

        def step(k, carry):
            for u in range(2):
                kk = 2 * k + u
                s = u
                s2 = 1 - u
                wait_gather(kk, s)
                pltpu.async_copy(
                    hubb[s], out_hbm.at[pl.ds(base + kk * NB, NB)], semst[s])

                @pl.when(kk + 1 < nblk)
                def _():
                    @pl.when(kk >= 1)
                    def _():
                        wait_store(kk - 1, s2)
                    issue(kk + 1, s2)
            return carry

        lax.fori_loop(0, nblk // 2, step, 0)
        wait_store(nblk - 2, 0)
        wait_store(nblk - 1, 1)

    return sck


def _combine_body(hubg_ref, m2p_ref, f0_ref, uv_ref, en_ref, spkf_ref,
                  wp_ref, we_ref, ws_ref, out_ref):
    m = m2p_ref[0, 0]
    f0 = f0_ref[0, 0]
    uv = uv_ref[0, 0]
    en = en_ref[0, 0]
    T = m.shape[0]

    f0d = jnp.exp2(f0)
    f0d = jnp.where(uv > 0, 0.0, f0d)
    f0d = jnp.where(m == 0, 0.0, f0d)
    f0_mel = 1127.0 * jnp.log(1.0 + f0d / 700.0)
    f0_mel = jnp.where(
        f0_mel > 0,
        (f0_mel - F0_MEL_MIN) * (F0_BIN - 2) / (F0_MEL_MAX - F0_MEL_MIN) + 1.0,
        f0_mel)
    f0_mel = jnp.where(f0_mel <= 1.0, 1.0, f0_mel)
    f0_mel = jnp.where(f0_mel > F0_BIN - 1, float(F0_BIN - 1), f0_mel)
    p = (f0_mel + 0.5).astype(jnp.int32)
    e = jnp.minimum(jnp.floor(en * (256.0 / 4.0)), 255.0).astype(jnp.int32)

    pioto = lax.broadcasted_iota(jnp.int32, (T, NP_PAD), 1)
    ponehot = (pioto == p[:, None]).astype(jnp.float32)
    eioto = lax.broadcasted_iota(jnp.int32, (T, 256), 1)
    eonehot = (eioto == e[:, None]).astype(jnp.float32)
    sioto = lax.broadcasted_iota(jnp.int32, (1, NS_PAD), 1)
    sonehot = (sioto == spkf_ref[0, 0, 0].astype(jnp.int32)).astype(
        jnp.float32)

    pe = jnp.dot(ponehot, wp_ref[...], preferred_element_type=jnp.float32)
    pe = pe + jnp.dot(eonehot, we_ref[...], preferred_element_type=jnp.float32)
    spkrow = jnp.dot(sonehot, ws_ref[...], preferred_element_type=jnp.float32)

    mask = (m > 0).astype(jnp.float32)[:, None]
    out_ref[0] = (hubg_ref[0] + pe + spkrow) * mask


def _combine(hubg, mel2ph, f0, uv, energy, spkf, wp_pad, W_energy, ws_pad,
             TBLK=512):
    B, T_mel, H = hubg.shape
    nt = T_mel // TBLK
    grid = (B, nt)
    rows3 = lambda x: x.reshape(B * nt, 1, TBLK)
    row_spec = pl.BlockSpec((1, 1, TBLK), lambda b, t: (b * nt + t, 0, 0))
    return pl.pallas_call(
        _combine_body,
        grid=grid,
        in_specs=[
            pl.BlockSpec((1, TBLK, H), lambda b, t: (b, t, 0)),
            row_spec, row_spec, row_spec, row_spec,
            pl.BlockSpec((1, 1, 1), lambda b, t: (b, 0, 0)),
            pl.BlockSpec((NP_PAD, H), lambda b, t: (0, 0)),
            pl.BlockSpec((256, H), lambda b, t: (0, 0)),
            pl.BlockSpec((NS_PAD, H), lambda b, t: (0, 0)),
        ],
        out_specs=pl.BlockSpec((1, TBLK, H), lambda b, t: (b, t, 0)),
        out_shape=jax.ShapeDtypeStruct((B, T_mel, H), jnp.float32),
    )(hubg, rows3(mel2ph), rows3(f0), rows3(uv), rows3(energy),
      spkf.reshape(B, 1, 1), wp_pad, W_energy, ws_pad)


def kernel(hubert, mel2ph, spk_embed, f0, uv, energy, W_spk, W_pitch, W_energy):
    B, T_ph, H = hubert.shape
    T_mel = mel2ph.shape[1]
    rows_total = B * T_mel
    hub2d = hubert.reshape(B * T_ph, H)

    sck = _make_sc_gather(rows_total, H, T_mel, T_ph, 32)
    hubg = sck(mel2ph.reshape(-1), hub2d).reshape(B, T_mel, H)

    wp_pad = jnp.pad(W_pitch, ((0, NP_PAD - W_pitch.shape[0]), (0, 0)))
    ws_pad = jnp.pad(W_spk, ((0, NS_PAD - W_spk.shape[0]), (0, 0)))
    spkf = spk_embed.astype(jnp.float32).reshape(B, 1)
    return _combine(hubg, mel2ph, f0, uv, energy, spkf, wp_pad, W_energy,
                    ws_pad)


# speedup vs baseline: 17.0018x; 1.1887x over previous
import functools
import math

import jax
import jax.numpy as jnp
from jax import lax
from jax.experimental import pallas as pl
from jax.experimental.pallas import tpu as pltpu
from jax.experimental.pallas import tpu_sc as plsc

F0_BIN = 256
F0_MEL_MIN = 1127.0 * math.log(1.0 + 50.0 / 700.0)
F0_MEL_MAX = 1127.0 * math.log(1.0 + 1100.0 / 700.0)

NP_PAD = 384
NS_PAD = 128
NTAB = NP_PAD + 256 + NS_PAD


@functools.lru_cache(maxsize=None)
def _make_sc_gather(rows_total, H, T_mel, T_ph, NW):
    rpw = rows_total // NW
    NB = 128
    nblk = rpw // NB
    mesh = plsc.VectorSubcoreMesh(core_axis_name="c", subcore_axis_name="s")

    @functools.partial(
        pl.kernel,
        out_type=jax.ShapeDtypeStruct((rows_total, H), jnp.float32),
        mesh=mesh,
        scratch_types=[
            pltpu.VMEM((rpw,), jnp.int32),
            [pltpu.VMEM((NB, H), jnp.float32) for _ in range(2)],
            [pltpu.SemaphoreType.DMA for _ in range(2)],
            [pltpu.SemaphoreType.DMA for _ in range(2)],
        ],
    )
    def sck(m2p_hbm, hub_hbm, out_hbm, gall, hubb, semg, semst):
        wid = lax.axis_index("s") * 2 + lax.axis_index("c")
        base = wid * rpw
        boff = (base // T_mel) * T_ph
        pltpu.sync_copy(m2p_hbm.at[pl.ds(base, rpw)], gall)

        @plsc.parallel_loop(0, rpw // 16, step=1, unroll=4)
        def fix(i):
            sl = pl.ds(i * 16, 16)
            m = gall[sl]
            gall[sl] = jnp.maximum(m - 1, 0) + boff

        def issue(k, s):
            pltpu.async_copy(hub_hbm.at[gall.at[pl.ds(k * NB, NB)]],
                             hubb[s], semg[s])

        def wait_gather(k, s):
            pltpu.make_async_copy(hub_hbm.at[gall.at[pl.ds(k * NB, NB)]],
                                  hubb[s], semg[s]).wait()

        def wait_store(k, s):
            pltpu.make_async_copy(
                hubb[s], out_hbm.at[pl.ds(base + k * NB, NB)], semst[s]).wait()

        issue(0, 0)

        def step(k, carry):
            for u in range(2):
                kk = 2 * k + u
                s = u
                s2 = 1 - u
                wait_gather(kk, s)
                pltpu.async_copy(
                    hubb[s], out_hbm.at[pl.ds(base + kk * NB, NB)], semst[s])

                @pl.when(kk + 1 < nblk)
                def _():
                    @pl.when(kk >= 1)
                    def _():
                        wait_store(kk - 1, s2)
                    issue(kk + 1, s2)
            return carry

        lax.fori_loop(0, nblk // 2, step, 0)
        wait_store(nblk - 2, 0)
        wait_store(nblk - 1, 1)

    return sck


def _combine_body(hubg_ref, m2p_ref, f0_ref, uv_ref, en_ref, spkf_ref,
                  tab_ref, out_ref):
    m = m2p_ref[0, 0]
    f0 = f0_ref[0, 0]
    uv = uv_ref[0, 0]
    en = en_ref[0, 0]
    T = m.shape[0]

    f0d = jnp.exp2(f0)
    f0d = jnp.where(uv > 0, 0.0, f0d)
    f0d = jnp.where(m == 0, 0.0, f0d)
    f0_mel = 1127.0 * jnp.log(1.0 + f0d / 700.0)
    f0_mel = jnp.where(
        f0_mel > 0,
        (f0_mel - F0_MEL_MIN) * (F0_BIN - 2) / (F0_MEL_MAX - F0_MEL_MIN) + 1.0,
        f0_mel)
    f0_mel = jnp.where(f0_mel <= 1.0, 1.0, f0_mel)
    f0_mel = jnp.where(f0_mel > F0_BIN - 1, float(F0_BIN - 1), f0_mel)
    p = (f0_mel + 0.5).astype(jnp.int32)
    e = jnp.minimum(jnp.floor(en * (256.0 / 4.0)), 255.0).astype(jnp.int32)

    ioto = lax.broadcasted_iota(jnp.int32, (T, NTAB), 1)
    s = spkf_ref[0, 0, 0].astype(jnp.int32) + (NP_PAD + 256)
    onehot = ((ioto == p[:, None]) | (ioto == (e + NP_PAD)[:, None])
              | (ioto == s)).astype(jnp.float32)
    emb = jnp.dot(onehot, tab_ref[...], preferred_element_type=jnp.float32)

    mask = (m > 0).astype(jnp.float32)[:, None]
    out_ref[0] = (hubg_ref[0] + emb) * mask


def _combine(hubg, mel2ph, f0, uv, energy, spkf, tab, TBLK=1024):
    B, T_mel, H = hubg.shape
    nt = T_mel // TBLK
    grid = (B, nt)
    rows3 = lambda x: x.reshape(B * nt, 1, TBLK)
    row_spec = pl.BlockSpec((1, 1, TBLK), lambda b, t: (b * nt + t, 0, 0))
    return pl.pallas_call(
        _combine_body,
        grid=grid,
        in_specs=[
            pl.BlockSpec((1, TBLK, H), lambda b, t: (b, t, 0)),
            row_spec, row_spec, row_spec, row_spec,
            pl.BlockSpec((1, 1, 1), lambda b, t: (b, 0, 0)),
            pl.BlockSpec((NTAB, H), lambda b, t: (0, 0)),
        ],
        out_specs=pl.BlockSpec((1, TBLK, H), lambda b, t: (b, t, 0)),
        out_shape=jax.ShapeDtypeStruct((B, T_mel, H), jnp.float32),
    )(hubg, rows3(mel2ph), rows3(f0), rows3(uv), rows3(energy),
      spkf.reshape(B, 1, 1), tab)


def kernel(hubert, mel2ph, spk_embed, f0, uv, energy, W_spk, W_pitch, W_energy):
    B, T_ph, H = hubert.shape
    T_mel = mel2ph.shape[1]
    rows_total = B * T_mel
    hub2d = hubert.reshape(B * T_ph, H)

    sck = _make_sc_gather(rows_total, H, T_mel, T_ph, 32)
    hubg = sck(mel2ph.reshape(-1), hub2d).reshape(B, T_mel, H)

    wp_pad = jnp.pad(W_pitch, ((0, NP_PAD - W_pitch.shape[0]), (0, 0)))
    ws_pad = jnp.pad(W_spk, ((0, NS_PAD - W_spk.shape[0]), (0, 0)))
    tab = jnp.concatenate([wp_pad, W_energy, ws_pad], axis=0)
    spkf = spk_embed.astype(jnp.float32).reshape(B, 1)
    return _combine(hubg, mel2ph, f0, uv, energy, spkf, tab)


# speedup vs baseline: 19.8884x; 1.1698x over previous
import functools
import math

import jax
import jax.numpy as jnp
from jax import lax
from jax.experimental import pallas as pl
from jax.experimental.pallas import tpu as pltpu
from jax.experimental.pallas import tpu_sc as plsc

F0_BIN = 256
F0_MEL_MIN = 1127.0 * math.log(1.0 + 50.0 / 700.0)
F0_MEL_MAX = 1127.0 * math.log(1.0 + 1100.0 / 700.0)

NP_TAB = 256
NS_PAD = 128
NTAB = NP_TAB + 256 + NS_PAD


@functools.lru_cache(maxsize=None)
def _make_sc_gather(rows_total, H, T_mel, T_ph, NW):
    rpw = rows_total // NW
    NB = 128
    nblk = rpw // NB
    mesh = plsc.VectorSubcoreMesh(core_axis_name="c", subcore_axis_name="s")

    @functools.partial(
        pl.kernel,
        out_type=jax.ShapeDtypeStruct((rows_total, H), jnp.float32),
        mesh=mesh,
        scratch_types=[
            pltpu.VMEM((rpw,), jnp.int32),
            [pltpu.VMEM((NB, H), jnp.float32) for _ in range(2)],
            [pltpu.SemaphoreType.DMA for _ in range(2)],
            [pltpu.SemaphoreType.DMA for _ in range(2)],
        ],
    )
    def sck(m2p_hbm, hub_hbm, out_hbm, gall, hubb, semg, semst):
        wid = lax.axis_index("s") * 2 + lax.axis_index("c")
        base = wid * rpw
        boff = (base // T_mel) * T_ph
        pltpu.sync_copy(m2p_hbm.at[pl.ds(base, rpw)], gall)

        @plsc.parallel_loop(0, rpw // 16, step=1, unroll=4)
        def fix(i):
            sl = pl.ds(i * 16, 16)
            m = gall[sl]
            gall[sl] = jnp.maximum(m - 1, 0) + boff

        def issue(k, s):
            pltpu.async_copy(hub_hbm.at[gall.at[pl.ds(k * NB, NB)]],
                             hubb[s], semg[s])

        def wait_gather(k, s):
            pltpu.make_async_copy(hub_hbm.at[gall.at[pl.ds(k * NB, NB)]],
                                  hubb[s], semg[s]).wait()

        def wait_store(k, s):
            pltpu.make_async_copy(
                hubb[s], out_hbm.at[pl.ds(base + k * NB, NB)], semst[s]).wait()

        issue(0, 0)

        def step(k, carry):
            for u in range(2):
                kk = 2 * k + u
                s = u
                s2 = 1 - u
                wait_gather(kk, s)
                pltpu.async_copy(
                    hubb[s], out_hbm.at[pl.ds(base + kk * NB, NB)], semst[s])

                @pl.when(kk + 1 < nblk)
                def _():
                    @pl.when(kk >= 1)
                    def _():
                        wait_store(kk - 1, s2)
                    issue(kk + 1, s2)
            return carry

        lax.fori_loop(0, nblk // 2, step, 0)
        wait_store(nblk - 2, 0)
        wait_store(nblk - 1, 1)

    return sck


def _combine_body(hubg_ref, m2p_ref, f0_ref, uv_ref, en_ref, spkf_ref,
                  tab_ref, out_ref):
    m = m2p_ref[0, 0]
    f0 = f0_ref[0, 0]
    uv = uv_ref[0, 0]
    en = en_ref[0, 0]
    T = m.shape[0]

    f0d = jnp.exp2(f0)
    f0d = jnp.where(uv > 0, 0.0, f0d)
    f0d = jnp.where(m == 0, 0.0, f0d)
    f0_mel = 1127.0 * jnp.log(1.0 + f0d / 700.0)
    f0_mel = jnp.where(
        f0_mel > 0,
        (f0_mel - F0_MEL_MIN) * (F0_BIN - 2) / (F0_MEL_MAX - F0_MEL_MIN) + 1.0,
        f0_mel)
    f0_mel = jnp.where(f0_mel <= 1.0, 1.0, f0_mel)
    f0_mel = jnp.where(f0_mel > F0_BIN - 1, float(F0_BIN - 1), f0_mel)
    p = (f0_mel + 0.5).astype(jnp.int32)
    e = jnp.minimum(jnp.floor(en * (256.0 / 4.0)), 255.0).astype(jnp.int32)

    ioto = lax.broadcasted_iota(jnp.int32, (T, NTAB), 1)
    s = spkf_ref[0, 0, 0].astype(jnp.int32) + (NP_TAB + 256)
    onehot = ((ioto == p[:, None]) | (ioto == (e + NP_TAB)[:, None])
              | (ioto == s)).astype(jnp.float32)
    emb = jnp.dot(onehot, tab_ref[...], preferred_element_type=jnp.float32)

    mask = (m > 0).astype(jnp.float32)[:, None]
    out_ref[0] = (hubg_ref[0] + emb) * mask


def _combine(hubg, mel2ph, f0, uv, energy, spkf, tab, TBLK=2048):
    B, T_mel, H = hubg.shape
    nt = T_mel // TBLK
    grid = (B, nt)
    rows3 = lambda x: x.reshape(B * nt, 1, TBLK)
    row_spec = pl.BlockSpec((1, 1, TBLK), lambda b, t: (b * nt + t, 0, 0))
    return pl.pallas_call(
        _combine_body,
        grid=grid,
        in_specs=[
            pl.BlockSpec((1, TBLK, H), lambda b, t: (b, t, 0)),
            row_spec, row_spec, row_spec, row_spec,
            pl.BlockSpec((1, 1, 1), lambda b, t: (b, 0, 0)),
            pl.BlockSpec((NTAB, H), lambda b, t: (0, 0)),
        ],
        out_specs=pl.BlockSpec((1, TBLK, H), lambda b, t: (b, t, 0)),
        out_shape=jax.ShapeDtypeStruct((B, T_mel, H), jnp.float32),
    )(hubg, rows3(mel2ph), rows3(f0), rows3(uv), rows3(energy),
      spkf.reshape(B, 1, 1), tab)


def kernel(hubert, mel2ph, spk_embed, f0, uv, energy, W_spk, W_pitch, W_energy):
    B, T_ph, H = hubert.shape
    T_mel = mel2ph.shape[1]
    rows_total = B * T_mel
    hub2d = hubert.reshape(B * T_ph, H)

    sck = _make_sc_gather(rows_total, H, T_mel, T_ph, 32)
    hubg = sck(mel2ph.reshape(-1), hub2d).reshape(B, T_mel, H)

    ws_pad = jnp.pad(W_spk, ((0, NS_PAD - W_spk.shape[0]), (0, 0)))
    tab = jnp.concatenate([W_pitch[:NP_TAB], W_energy, ws_pad], axis=0)
    spkf = spk_embed.astype(jnp.float32).reshape(B, 1)
    return _combine(hubg, mel2ph, f0, uv, energy, spkf, tab)


# speedup vs baseline: 21.4518x; 1.0786x over previous
import functools
import math

import jax
import jax.numpy as jnp
from jax import lax
from jax.experimental import pallas as pl
from jax.experimental.pallas import tpu as pltpu
from jax.experimental.pallas import tpu_sc as plsc

F0_BIN = 256
F0_MEL_MIN = 1127.0 * math.log(1.0 + 50.0 / 700.0)
F0_MEL_MAX = 1127.0 * math.log(1.0 + 1100.0 / 700.0)

NP_TAB = 256
NS_PAD = 128
NTAB = NP_TAB + 256 + NS_PAD


@functools.lru_cache(maxsize=None)
def _make_sc_gather(rows_total, H, T_mel, T_ph, NW):
    rpw = rows_total // NW
    NB = 128
    nblk = rpw // NB
    mesh = plsc.VectorSubcoreMesh(core_axis_name="c", subcore_axis_name="s")

    @functools.partial(
        pl.kernel,
        out_type=jax.ShapeDtypeStruct((rows_total, H), jnp.float32),
        mesh=mesh,
        scratch_types=[
            pltpu.VMEM((rpw,), jnp.int32),
            [pltpu.VMEM((NB, H), jnp.float32) for _ in range(2)],
            [pltpu.SemaphoreType.DMA for _ in range(2)],
            [pltpu.SemaphoreType.DMA for _ in range(2)],
        ],
    )
    def sck(m2p_hbm, hub_hbm, out_hbm, gall, hubb, semg, semst):
        wid = lax.axis_index("s") * 2 + lax.axis_index("c")
        base = wid * rpw
        boff = (base // T_mel) * T_ph
        pltpu.sync_copy(m2p_hbm.at[pl.ds(base, rpw)], gall)

        @plsc.parallel_loop(0, rpw // 16, step=1, unroll=4)
        def fix(i):
            sl = pl.ds(i * 16, 16)
            m = gall[sl]
            gall[sl] = jnp.maximum(m - 1, 0) + boff

        def issue(k, s):
            pltpu.async_copy(hub_hbm.at[gall.at[pl.ds(k * NB, NB)]],
                             hubb[s], semg[s])

        def wait_gather(k, s):
            pltpu.make_async_copy(hub_hbm.at[gall.at[pl.ds(k * NB, NB)]],
                                  hubb[s], semg[s]).wait()

        def wait_store(k, s):
            pltpu.make_async_copy(
                hubb[s], out_hbm.at[pl.ds(base + k * NB, NB)], semst[s]).wait()

        issue(0, 0)

        def step(k, carry):
            for u in range(2):
                kk = 2 * k + u
                s = u
                s2 = 1 - u
                wait_gather(kk, s)
                pltpu.async_copy(
                    hubb[s], out_hbm.at[pl.ds(base + kk * NB, NB)], semst[s])

                @pl.when(kk + 1 < nblk)
                def _():
                    @pl.when(kk >= 1)
                    def _():
                        wait_store(kk - 1, s2)
                    issue(kk + 1, s2)
            return carry

        lax.fori_loop(0, nblk // 2, step, 0)
        wait_store(nblk - 2, 0)
        wait_store(nblk - 1, 1)

    return sck


def _combine_body(hubg_ref, m2p_ref, f0_ref, uv_ref, en_ref, spkf_ref,
                  tab_ref, out_ref):
    m = m2p_ref[0, 0]
    f0 = f0_ref[0, 0]
    uv = uv_ref[0, 0]
    en = en_ref[0, 0]
    T = m.shape[0]

    f0d = jnp.exp2(f0)
    f0d = jnp.where(uv > 0, 0.0, f0d)
    f0d = jnp.where(m == 0, 0.0, f0d)
    f0_mel = 1127.0 * jnp.log(1.0 + f0d / 700.0)
    f0_mel = jnp.where(
        f0_mel > 0,
        (f0_mel - F0_MEL_MIN) * (F0_BIN - 2) / (F0_MEL_MAX - F0_MEL_MIN) + 1.0,
        f0_mel)
    f0_mel = jnp.where(f0_mel <= 1.0, 1.0, f0_mel)
    f0_mel = jnp.where(f0_mel > F0_BIN - 1, float(F0_BIN - 1), f0_mel)
    p = (f0_mel + 0.5).astype(jnp.int32)
    e = jnp.minimum(jnp.floor(en * (256.0 / 4.0)), 255.0).astype(jnp.int32)

    ioto = lax.broadcasted_iota(jnp.int32, (T, NTAB), 1)
    s = spkf_ref[0, 0, 0].astype(jnp.int32) + (NP_TAB + 256)
    onehot = ((ioto == p[:, None]) | (ioto == (e + NP_TAB)[:, None])
              | (ioto == s)).astype(jnp.float32)
    emb = jnp.dot(onehot, tab_ref[...], preferred_element_type=jnp.float32)

    mask = (m > 0).astype(jnp.float32)[:, None]
    out_ref[0] = (hubg_ref[0] + emb) * mask


def _combine(hubg, mel2ph, f0, uv, energy, spkf, tab, TBLK=4096):
    B, T_mel, H = hubg.shape
    nt = T_mel // TBLK
    grid = (B, nt)
    rows3 = lambda x: x.reshape(B * nt, 1, TBLK)
    row_spec = pl.BlockSpec((1, 1, TBLK), lambda b, t: (b * nt + t, 0, 0))
    return pl.pallas_call(
        _combine_body,
        grid=grid,
        in_specs=[
            pl.BlockSpec((1, TBLK, H), lambda b, t: (b, t, 0)),
            row_spec, row_spec, row_spec, row_spec,
            pl.BlockSpec((1, 1, 1), lambda b, t: (b, 0, 0)),
            pl.BlockSpec((NTAB, H), lambda b, t: (0, 0)),
        ],
        out_specs=pl.BlockSpec((1, TBLK, H), lambda b, t: (b, t, 0)),
        out_shape=jax.ShapeDtypeStruct((B, T_mel, H), jnp.float32),
    )(hubg, rows3(mel2ph), rows3(f0), rows3(uv), rows3(energy),
      spkf.reshape(B, 1, 1), tab)


def kernel(hubert, mel2ph, spk_embed, f0, uv, energy, W_spk, W_pitch, W_energy):
    B, T_ph, H = hubert.shape
    T_mel = mel2ph.shape[1]
    rows_total = B * T_mel
    hub2d = hubert.reshape(B * T_ph, H)

    sck = _make_sc_gather(rows_total, H, T_mel, T_ph, 32)
    hubg = sck(mel2ph.reshape(-1), hub2d).reshape(B, T_mel, H)

    ws_pad = jnp.pad(W_spk, ((0, NS_PAD - W_spk.shape[0]), (0, 0)))
    tab = jnp.concatenate([W_pitch[:NP_TAB], W_energy, ws_pad], axis=0)
    spkf = spk_embed.astype(jnp.float32).reshape(B, 1)
    return _combine(hubg, mel2ph, f0, uv, energy, spkf, tab)
